# baseline (speedup 1.0000x reference)
import functools

import jax
import jax.numpy as jnp
from jax import lax
from jax.experimental import pallas as pl
from jax.experimental.pallas import tpu as pltpu
from jax.experimental.pallas import tpu_sc as plsc

_N_NODES = 10000
_N_EDGES = 320000
_DE = 16
_G = 128
_NG = _N_EDGES // _G
_NW = 32
_GPT = _NG // _NW
_CH = 6
_E_CH = _CH * _G
_CPT = _GPT // _CH
_NPAIR = _CPT // 2
_RPT = _N_NODES // 16


def _sc_scatter(ea_t, idx3):
    mesh = plsc.VectorSubcoreMesh(core_axis_name="c", subcore_axis_name="s")
    i32 = jnp.int32

    @functools.partial(
        pl.kernel,
        out_type=[
            jax.ShapeDtypeStruct((2, _N_NODES, _DE), jnp.float32),
            jax.ShapeDtypeStruct((_N_EDGES,), jnp.float32),
        ],
        mesh=mesh,
        compiler_params=pltpu.CompilerParams(use_tc_tiling_on_sc=False,
                                             needs_layout_passes=False),
        scratch_types=[
            pltpu.VMEM_SHARED((_N_NODES, _DE), jnp.float32),
            pltpu.VMEM((_DE, _E_CH), jnp.float32),
            pltpu.VMEM((_DE, _E_CH), jnp.float32),
            pltpu.VMEM((_CH, 1, _G), jnp.int32),
            pltpu.VMEM((_CH, 1, _G), jnp.int32),
            pltpu.VMEM((_E_CH, _DE), jnp.float32),
            pltpu.VMEM((_E_CH, _DE), jnp.float32),
            pltpu.VMEM((_E_CH,), jnp.float32),
            pltpu.VMEM((_E_CH,), jnp.float32),
            pltpu.SemaphoreType.DMA,
            pltpu.SemaphoreType.DMA,
            pltpu.SemaphoreType.DMA,
            pltpu.SemaphoreType.DMA,
            pltpu.SemaphoreType.DMA,
            pltpu.SemaphoreType.DMA,
        ],
    )
    def k(ea_hbm, ei_hbm, part_hbm, out2_hbm, acc,
          cols_a, cols_b, idx_a, idx_b, rows_a, rows_b,
          o2_a, o2_b, sem_la, sem_lb, sem_sa, sem_sb, sem_oa, sem_ob):
        c = lax.axis_index("c").astype(i32)
        s = lax.axis_index("s").astype(i32)
        wid = s * i32(2) + c
        g0 = wid * i32(_GPT)

        zf = jnp.zeros((16,), jnp.float32)

        def zblk(r, carry):
            rows_a[r, :] = zf
            return carry

        lax.fori_loop(i32(0), i32(_RPT), zblk, i32(0))
        pltpu.sync_copy(rows_a.at[pl.ds(0, _RPT)],
                        acc.at[pl.ds(s * i32(_RPT), _RPT)])
        plsc.subcore_barrier()

        lane = lax.iota(jnp.int32, 16)
        rots = [jnp.bitwise_and(lane + i32(kk), i32(15)) for kk in range(16)]

        def _start_loads(g, colsb, idxb, sem):
            e0 = g * i32(_G)
            pltpu.async_copy(ea_hbm.at[:, pl.ds(e0, _E_CH)], colsb, sem)
            pltpu.async_copy(ei_hbm.at[pl.ds(g, _CH), pl.ds(1, 1), :],
                             idxb, sem)

        def _wait_loads(g, colsb, idxb, sem):
            e0 = g * i32(_G)
            pltpu.make_async_copy(ea_hbm.at[:, pl.ds(e0, _E_CH)], colsb,
                                  sem).wait()
            pltpu.make_async_copy(ei_hbm.at[pl.ds(g, _CH), pl.ds(1, 1), :],
                                  idxb, sem).wait()

        def _transpose(colsb, rowsb, nblk):
            def blk(b, carry):
                b16 = b * i32(16)
                eidxs = [rots[kk] + b16 for kk in range(16)]
                xs = [plsc.load_gather(colsb, [lane, eidxs[kk]])
                      for kk in range(16)]
                for kk in range(16):
                    plsc.store_scatter(rowsb, [eidxs[kk], lane], xs[kk])
                return carry

            lax.fori_loop(i32(0), i32(nblk), blk, i32(0))

        def _copy_out2(colsb, o2b, nblk):
            def blk(b, carry):
                b16 = b * i32(16)
                o2b[pl.ds(b16, 16)] = colsb[i32(_DE - 1), pl.ds(b16, 16)]
                return carry

            lax.fori_loop(i32(0), i32(nblk), blk, i32(0))

        def _fire(g, colsb, idxb, rowsb, o2b, sem_s, sem_o):
            _transpose(colsb, rowsb, _E_CH // 16)
            _copy_out2(colsb, o2b, _E_CH // 16)
            for j in range(_CH):
                pltpu.async_copy(rowsb.at[pl.ds(j * _G, _G)],
                                 acc.at[idxb.at[i32(j), i32(0)]],
                                 sem_s, add=True)
            pltpu.async_copy(o2b, out2_hbm.at[pl.ds(g * i32(_G), _E_CH)],
                             sem_o)

        def _drain(g, rowsb, idxb, o2b, sem_s, sem_o):
            for j in range(_CH):
                pltpu.make_async_copy(rowsb.at[pl.ds(j * _G, _G)],
                                      acc.at[idxb.at[i32(j), i32(0)]],
                                      sem_s).wait()
            pltpu.make_async_copy(o2b,
                                  out2_hbm.at[pl.ds(g * i32(_G), _E_CH)],
                                  sem_o).wait()

        _start_loads(g0, cols_a, idx_a, sem_la)

        def pair_body(p, carry):
            ga = g0 + p * i32(2 * _CH)
            gb = ga + i32(_CH)
            _wait_loads(ga, cols_a, idx_a, sem_la)
            _start_loads(gb, cols_b, idx_b, sem_lb)

            @pl.when(p > i32(0))
            def _():
                _drain(ga, rows_a, idx_a, o2_a, sem_sa, sem_oa)

            _fire(ga, cols_a, idx_a, rows_a, o2_a, sem_sa, sem_oa)

            _wait_loads(gb, cols_b, idx_b, sem_lb)
            _start_loads(ga + i32(2 * _CH), cols_a, idx_a, sem_la)

            @pl.when(p > i32(0))
            def _():
                _drain(gb, rows_b, idx_b, o2_b, sem_sb, sem_ob)

            _fire(gb, cols_b, idx_b, rows_b, o2_b, sem_sb, sem_ob)
            return carry

        lax.fori_loop(i32(0), i32(_NPAIR), pair_body, i32(0))

        g12 = g0 + i32(12 * _CH)
        _wait_loads(g12, cols_a, idx_a, sem_la)
        _drain(g12, rows_a, idx_a, o2_a, sem_sa, sem_oa)
        _drain(g12, rows_b, idx_b, o2_b, sem_sb, sem_ob)
        _fire(g12, cols_a, idx_a, rows_a, o2_a, sem_sa, sem_oa)
        _drain(g12, rows_a, idx_a, o2_a, sem_sa, sem_oa)

        @pl.when(wid < i32(4))
        def _():
            gx = i32(_NW * _GPT) + wid
            ex = gx * i32(_G)
            pltpu.sync_copy(ea_hbm.at[:, pl.ds(ex, _G)],
                            cols_a.at[:, pl.ds(0, _G)])
            pltpu.sync_copy(ei_hbm.at[pl.ds(gx, 1), pl.ds(1, 1), :],
                            idx_a.at[pl.ds(0, 1)])
            _transpose(cols_a, rows_a, _G // 16)
            _copy_out2(cols_a, o2_a, _G // 16)
            pltpu.sync_copy(rows_a.at[pl.ds(0, _G)],
                            acc.at[idx_a.at[i32(0), i32(0)]], add=True)
            pltpu.sync_copy(o2_a.at[pl.ds(0, _G)],
                            out2_hbm.at[pl.ds(ex, _G)])

        plsc.subcore_barrier()
        pltpu.sync_copy(acc.at[pl.ds(s * i32(_RPT), _RPT)],
                        part_hbm.at[c].at[pl.ds(s * i32(_RPT), _RPT)])

    return k(ea_t, idx3)


def _combine(p_ref, o_ref):
    o_ref[...] = p_ref[0] + p_ref[1]


def kernel(x, edge_index, edge_attr, u, batch):
    ea_t = edge_attr.astype(jnp.float32).T
    idx3 = (lax.bitcast_convert_type(edge_index.astype(jnp.uint32), jnp.int32)
            .reshape(2, _NG, _G).transpose(1, 0, 2))
    part, out2 = _sc_scatter(ea_t, idx3)
    summed16 = pl.pallas_call(
        _combine,
        out_shape=jax.ShapeDtypeStruct((_N_NODES, _DE), jnp.float32),
    )(part)
    summed = summed16[:, : _DE - 1]
    return (summed, out2)


# speedup vs baseline: 1.1037x; 1.1037x over previous
import functools

import jax
import jax.numpy as jnp
from jax import lax
from jax.experimental import pallas as pl
from jax.experimental.pallas import tpu as pltpu
from jax.experimental.pallas import tpu_sc as plsc

_N_NODES = 10000
_N_EDGES = 320000
_DE = 16
_G = 128
_NG = _N_EDGES // _G
_NW = 32
_GPT = _NG // _NW
_CH = 6
_E_CH = _CH * _G
_CPT = _GPT // _CH
_NPAIR = _CPT // 2
_RPT = _N_NODES // 16


def _sc_scatter(ea_t, idx3):
    mesh = plsc.VectorSubcoreMesh(core_axis_name="c", subcore_axis_name="s")
    i32 = jnp.int32

    @functools.partial(
        pl.kernel,
        out_type=[
            jax.ShapeDtypeStruct((2, _N_NODES, _DE), jnp.float32),
            jax.ShapeDtypeStruct((_N_EDGES,), jnp.float32),
        ],
        mesh=mesh,
        compiler_params=pltpu.CompilerParams(use_tc_tiling_on_sc=False,
                                             needs_layout_passes=False),
        scratch_types=[
            pltpu.VMEM_SHARED((_N_NODES, _DE), jnp.float32),
            pltpu.VMEM((_DE, _E_CH), jnp.float32),
            pltpu.VMEM((_DE, _E_CH), jnp.float32),
            pltpu.VMEM((_CH, 1, _G), jnp.int32),
            pltpu.VMEM((_CH, 1, _G), jnp.int32),
            pltpu.VMEM((_E_CH, _DE), jnp.float32),
            pltpu.VMEM((_E_CH, _DE), jnp.float32),
            pltpu.VMEM((_E_CH,), jnp.float32),
            pltpu.VMEM((_E_CH,), jnp.float32),
            pltpu.SemaphoreType.DMA,
            pltpu.SemaphoreType.DMA,
            pltpu.SemaphoreType.DMA,
            pltpu.SemaphoreType.DMA,
            pltpu.SemaphoreType.DMA,
            pltpu.SemaphoreType.DMA,
        ],
    )
    def k(ea_hbm, ei_hbm, part_hbm, out2_hbm, acc,
          cols_a, cols_b, idx_a, idx_b, rows_a, rows_b,
          o2_a, o2_b, sem_la, sem_lb, sem_sa, sem_sb, sem_oa, sem_ob):
        c = lax.axis_index("c").astype(i32)
        s = lax.axis_index("s").astype(i32)
        wid = s * i32(2) + c
        g0 = wid * i32(_GPT)

        zf = jnp.zeros((16,), jnp.float32)

        def zblk(r, carry):
            rows_a[r, :] = zf
            return carry

        lax.fori_loop(i32(0), i32(_RPT), zblk, i32(0))
        pltpu.sync_copy(rows_a.at[pl.ds(0, _RPT)],
                        acc.at[pl.ds(s * i32(_RPT), _RPT)])
        plsc.subcore_barrier()

        lane = lax.iota(jnp.int32, 16)
        rots = [jnp.bitwise_and(lane + i32(kk), i32(15)) for kk in range(16)]

        def _start_loads(g, colsb, idxb, sem):
            e0 = g * i32(_G)
            pltpu.async_copy(ea_hbm.at[:, pl.ds(e0, _E_CH)], colsb, sem)
            pltpu.async_copy(ei_hbm.at[pl.ds(g, _CH), pl.ds(1, 1), :],
                             idxb, sem)

        def _wait_loads(g, colsb, idxb, sem):
            e0 = g * i32(_G)
            pltpu.make_async_copy(ea_hbm.at[:, pl.ds(e0, _E_CH)], colsb,
                                  sem).wait()
            pltpu.make_async_copy(ei_hbm.at[pl.ds(g, _CH), pl.ds(1, 1), :],
                                  idxb, sem).wait()

        def _transpose(colsb, rowsb, nblk):
            def blk(b, carry):
                b16 = b * i32(16)
                eidxs = [rots[kk] + b16 for kk in range(16)]
                xs = [plsc.load_gather(colsb, [lane, eidxs[kk]])
                      for kk in range(16)]
                for kk in range(16):
                    plsc.store_scatter(rowsb, [eidxs[kk], lane], xs[kk])
                return carry

            lax.fori_loop(i32(0), i32(nblk), blk, i32(0))

        def _copy_out2(colsb, o2b, nblk):
            def blk(b, carry):
                b16 = b * i32(16)
                o2b[pl.ds(b16, 16)] = colsb[i32(_DE - 1), pl.ds(b16, 16)]
                return carry

            lax.fori_loop(i32(0), i32(nblk), blk, i32(0))

        def _fire(g, colsb, idxb, rowsb, o2b, sem_s, sem_o):
            _transpose(colsb, rowsb, _E_CH // 16)
            _copy_out2(colsb, o2b, _E_CH // 16)
            for j in range(_CH):
                pltpu.async_copy(rowsb.at[pl.ds(j * _G, _G)],
                                 acc.at[idxb.at[i32(j), i32(0)]],
                                 sem_s, add=True)
            pltpu.async_copy(o2b, out2_hbm.at[pl.ds(g * i32(_G), _E_CH)],
                             sem_o)

        def _drain(g, rowsb, idxb, o2b, sem_s, sem_o):
            for j in range(_CH):
                pltpu.make_async_copy(rowsb.at[pl.ds(j * _G, _G)],
                                      acc.at[idxb.at[i32(j), i32(0)]],
                                      sem_s).wait()
            pltpu.make_async_copy(o2b,
                                  out2_hbm.at[pl.ds(g * i32(_G), _E_CH)],
                                  sem_o).wait()

        _start_loads(g0, cols_a, idx_a, sem_la)

        def pair_body(p, carry):
            ga = g0 + p * i32(2 * _CH)
            gb = ga + i32(_CH)
            _wait_loads(ga, cols_a, idx_a, sem_la)
            _start_loads(gb, cols_b, idx_b, sem_lb)

            @pl.when(p > i32(0))
            def _():
                _drain(ga, rows_a, idx_a, o2_a, sem_sa, sem_oa)

            _fire(ga, cols_a, idx_a, rows_a, o2_a, sem_sa, sem_oa)

            _wait_loads(gb, cols_b, idx_b, sem_lb)
            _start_loads(ga + i32(2 * _CH), cols_a, idx_a, sem_la)

            @pl.when(p > i32(0))
            def _():
                _drain(gb, rows_b, idx_b, o2_b, sem_sb, sem_ob)

            _fire(gb, cols_b, idx_b, rows_b, o2_b, sem_sb, sem_ob)
            return carry

        lax.fori_loop(i32(0), i32(_NPAIR), pair_body, i32(0))

        g12 = g0 + i32(12 * _CH)
        _wait_loads(g12, cols_a, idx_a, sem_la)
        _drain(g12, rows_a, idx_a, o2_a, sem_sa, sem_oa)
        _drain(g12, rows_b, idx_b, o2_b, sem_sb, sem_ob)
        _fire(g12, cols_a, idx_a, rows_a, o2_a, sem_sa, sem_oa)
        _drain(g12, rows_a, idx_a, o2_a, sem_sa, sem_oa)

        @pl.when(wid < i32(4))
        def _():
            gx = i32(_NW * _GPT) + wid
            ex = gx * i32(_G)
            pltpu.sync_copy(ea_hbm.at[:, pl.ds(ex, _G)],
                            cols_a.at[:, pl.ds(0, _G)])
            pltpu.sync_copy(ei_hbm.at[pl.ds(gx, 1), pl.ds(1, 1), :],
                            idx_a.at[pl.ds(0, 1)])
            _transpose(cols_a, rows_a, _G // 16)
            _copy_out2(cols_a, o2_a, _G // 16)
            pltpu.sync_copy(rows_a.at[pl.ds(0, _G)],
                            acc.at[idx_a.at[i32(0), i32(0)]], add=True)
            pltpu.sync_copy(o2_a.at[pl.ds(0, _G)],
                            out2_hbm.at[pl.ds(ex, _G)])

        plsc.subcore_barrier()
        pltpu.sync_copy(acc.at[pl.ds(s * i32(_RPT), _RPT)],
                        part_hbm.at[c].at[pl.ds(s * i32(_RPT), _RPT)])

    return k(ea_t, idx3)


def _combine(p_ref, o_ref):
    o_ref[...] = p_ref[0] + p_ref[1]


def kernel(x, edge_index, edge_attr, u, batch):
    ea_t = edge_attr.astype(jnp.float32).T
    idx3 = (lax.bitcast_convert_type(edge_index.astype(jnp.uint32), jnp.int32)
            .reshape(2, _NG, _G).transpose(1, 0, 2))
    part, out2 = _sc_scatter(ea_t, idx3)
    p = part.reshape(2, (_N_NODES * _DE) // 128, 128)
    summed16 = pl.pallas_call(
        _combine,
        out_shape=jax.ShapeDtypeStruct(((_N_NODES * _DE) // 128, 128),
                                       jnp.float32),
    )(p)
    summed = summed16.reshape(_N_NODES, _DE)[:, : _DE - 1]
    return (summed, out2)
